# baseline (speedup 1.0000x reference)
import jax
import jax.numpy as jnp
from jax.experimental import pallas as pl
from jax.experimental.pallas import tpu as pltpu

_BLK_N = 2048


def _add_kernel(x_ref, emb_ref, o_ref):
    o_ref[...] = x_ref[...] + emb_ref[...]


def kernel(x, emb):
    B, N, D = x.shape
    nb = N // _BLK_N
    return pl.pallas_call(
        _add_kernel,
        grid=(nb, B),
        in_specs=[
            pl.BlockSpec((1, _BLK_N, D), lambda i, b: (b, i, 0)),
            pl.BlockSpec((_BLK_N, D), lambda i, b: (i, 0)),
        ],
        out_specs=pl.BlockSpec((1, _BLK_N, D), lambda i, b: (b, i, 0)),
        out_shape=jax.ShapeDtypeStruct((B, N, D), x.dtype),
        compiler_params=pltpu.CompilerParams(
            dimension_semantics=("parallel", "parallel"),
        ),
    )(x, emb[:N])


# speedup vs baseline: 1.0256x; 1.0256x over previous
import jax
import jax.numpy as jnp
from jax.experimental import pallas as pl
from jax.experimental.pallas import tpu as pltpu

_BLK_N = 2048
_BLK_B = 2


def _add_kernel(x_ref, emb_ref, o_ref):
    o_ref[...] = x_ref[...] + emb_ref[...]


def kernel(x, emb):
    B, N, D = x.shape
    nb = N // _BLK_N
    return pl.pallas_call(
        _add_kernel,
        grid=(nb, B // _BLK_B),
        in_specs=[
            pl.BlockSpec((_BLK_B, _BLK_N, D), lambda i, b: (b, i, 0)),
            pl.BlockSpec((_BLK_N, D), lambda i, b: (i, 0)),
        ],
        out_specs=pl.BlockSpec((_BLK_B, _BLK_N, D), lambda i, b: (b, i, 0)),
        out_shape=jax.ShapeDtypeStruct((B, N, D), x.dtype),
        compiler_params=pltpu.CompilerParams(
            dimension_semantics=("parallel", "parallel"),
            vmem_limit_bytes=120 * 1024 * 1024,
        ),
    )(x, emb[:N])
